# speedup vs baseline: 15.3245x; 15.3245x over previous
import functools

import jax
import jax.numpy as jnp
from jax import lax
from jax.experimental import pallas as pl
from jax.experimental.pallas import tpu as pltpu
from jax.experimental.pallas import tpu_sc as plsc

N = 10000
D = 128
E = 320000

NC, NS = 2, 16
NW = NC * NS
NPAD = 10240
RPT = NPAD // NS
EPT = E // NW
K = 80
NCHUNK = EPT // K

_mesh = plsc.VectorSubcoreMesh(core_axis_name="c", subcore_axis_name="s")


@functools.partial(
    pl.kernel,
    out_type=jax.ShapeDtypeStruct((NC, NPAD), jnp.float32),
    mesh=_mesh,
    scratch_types=[
        pltpu.VMEM((K,), jnp.int32),
        pltpu.VMEM((K,), jnp.float32),
        pltpu.VMEM((RPT,), jnp.float32),
        pltpu.VMEM_SHARED((NPAD,), jnp.float32),
    ],
)
def _degree_kernel(src_hbm, out_hbm, idx_v, ones_v, zero_v, hist_sh):
    c = lax.axis_index("c")
    s = lax.axis_index("s")
    wid = s * NC + c

    for j in range(RPT // 16):
        zero_v[pl.ds(16 * j, 16)] = jnp.zeros((16,), jnp.float32)
    for j in range(K // 16):
        ones_v[pl.ds(16 * j, 16)] = jnp.ones((16,), jnp.float32)
    pltpu.sync_copy(zero_v, hist_sh.at[pl.ds(s * RPT, RPT)])
    plsc.subcore_barrier()

    base0 = wid * EPT

    def body(i, carry):
        base = pl.multiple_of(base0 + i * K, 8)
        pltpu.sync_copy(src_hbm.at[pl.ds(base, K)], idx_v)
        pltpu.sync_copy(ones_v, hist_sh.at[idx_v], add=True)
        return carry

    lax.fori_loop(0, NCHUNK, body, 0)
    plsc.subcore_barrier()

    pltpu.sync_copy(hist_sh.at[pl.ds(s * RPT, RPT)],
                    out_hbm.at[c, pl.ds(s * RPT, RPT)])


BR = 2000


def _scale_body(da_ref, db_ref, x_ref, xs_ref, r_ref):
    deg = jnp.maximum(da_ref[...] + db_ref[...], 1.0)
    r = lax.rsqrt(deg)
    r_ref[...] = r
    xs_ref[...] = x_ref[...] * r


def _scale_call(da, db, x):
    return pl.pallas_call(
        _scale_body,
        grid=(N // BR,),
        in_specs=[
            pl.BlockSpec((BR, 1), lambda i: (i, 0)),
            pl.BlockSpec((BR, 1), lambda i: (i, 0)),
            pl.BlockSpec((BR, D), lambda i: (i, 0)),
        ],
        out_specs=[
            pl.BlockSpec((BR, D), lambda i: (i, 0)),
            pl.BlockSpec((BR, 1), lambda i: (i, 0)),
        ],
        out_shape=[
            jax.ShapeDtypeStruct((N, D), jnp.float32),
            jax.ShapeDtypeStruct((N, 1), jnp.float32),
        ],
    )(da, db, x)


@functools.partial(
    pl.kernel,
    out_type=jax.ShapeDtypeStruct((NC, NPAD, D), jnp.float32),
    mesh=_mesh,
    scratch_types=[
        pltpu.VMEM((K,), jnp.int32),
        pltpu.VMEM((K,), jnp.int32),
        pltpu.VMEM((K, D), jnp.float32),
        pltpu.VMEM_SHARED((NPAD, D), jnp.float32),
        pltpu.SemaphoreType.DMA,
    ],
)
def _agg_kernel(xs_hbm, src_hbm, dst_hbm, zeros_hbm, out_hbm,
                sidx_v, didx_v, rows_v, agg_sh, sem):
    c = lax.axis_index("c")
    s = lax.axis_index("s")
    wid = s * NC + c
    r0 = s * RPT

    pltpu.sync_copy(zeros_hbm.at[pl.ds(r0, RPT)], agg_sh.at[pl.ds(r0, RPT)])
    plsc.subcore_barrier()

    base0 = wid * EPT

    def body(i, carry):
        base = pl.multiple_of(base0 + i * K, 8)
        pltpu.sync_copy(src_hbm.at[pl.ds(base, K)], sidx_v)
        pltpu.sync_copy(dst_hbm.at[pl.ds(base, K)], didx_v)
        pltpu.async_copy(xs_hbm.at[sidx_v], rows_v, sem).wait()
        pltpu.sync_copy(rows_v, agg_sh.at[didx_v], add=True)
        return carry

    lax.fori_loop(0, NCHUNK, body, 0)
    plsc.subcore_barrier()

    pltpu.sync_copy(agg_sh.at[pl.ds(r0, RPT)], out_hbm.at[c, pl.ds(r0, RPT)])


def _ffn_body(p0_ref, p1_ref, r_ref, w1_ref, b1_ref, w2_ref, b2_ref, out_ref):
    a = (p0_ref[...] + p1_ref[...]) * r_ref[...]
    h = jnp.maximum(jnp.dot(a, w1_ref[...],
                            preferred_element_type=jnp.float32) + b1_ref[...],
                    0.0)
    out_ref[...] = jnp.dot(h, w2_ref[...],
                           preferred_element_type=jnp.float32) + b2_ref[...]


def _ffn_call(p0, p1, r, w1, b1, w2, b2):
    full = lambda i: (0, 0)
    return pl.pallas_call(
        _ffn_body,
        grid=(N // BR,),
        in_specs=[
            pl.BlockSpec((BR, D), lambda i: (i, 0)),
            pl.BlockSpec((BR, D), lambda i: (i, 0)),
            pl.BlockSpec((BR, 1), lambda i: (i, 0)),
            pl.BlockSpec((D, D), full),
            pl.BlockSpec((1, D), full),
            pl.BlockSpec((D, D), full),
            pl.BlockSpec((1, D), full),
        ],
        out_specs=pl.BlockSpec((BR, D), lambda i: (i, 0)),
        out_shape=jax.ShapeDtypeStruct((N, D), jnp.float32),
    )(p0, p1, r, w1, b1, w2, b2)


def kernel(x, edge_index, W1, b1, W2, b2):
    src = edge_index[0]
    dst = edge_index[1]

    deg_p = _degree_kernel(src)
    da = deg_p[0].reshape(NPAD, 1)
    db = deg_p[1].reshape(NPAD, 1)
    xs, r = _scale_call(da, db, x)

    zeros = jnp.zeros((NPAD, D), jnp.float32)
    agg_p = _agg_kernel(xs, src, dst, zeros)

    return _ffn_call(agg_p[0], agg_p[1], r,
                     W1, b1.reshape(1, D), W2, b2.reshape(1, D))
